# baseline (speedup 1.0000x reference)
import functools

import jax
import jax.numpy as jnp
from jax import lax
from jax.experimental import pallas as pl
from jax.experimental.pallas import tpu as pltpu
from jax.experimental.pallas import tpu_sc as plsc

B, L, K = 1024, 200, 3
D = 64
V = 1000518
NC, NS = 2, 16
NW = NC * NS
B_PER_W = B // NW


CB = 1024
G = -(-V // CB)
VP = G * CB


def _tc_detile(weT):
    def body(in_ref, out_ref):
        row = lax.broadcasted_iota(jnp.int32, (D, D), 0)
        col = lax.broadcasted_iota(jnp.int32, (D, D), 1)
        eye = jnp.where(row == col, 1.0, 0.0).astype(jnp.float32)
        out_ref[:, 0:D] = lax.dot_general(
            in_ref[...], eye, (((0,), (0,)), ((), ())),
            precision=lax.Precision.HIGHEST,
            preferred_element_type=jnp.float32)

    return pl.pallas_call(
        body,
        grid=(G,),
        in_specs=[pl.BlockSpec((D, CB), lambda g: (0, g))],
        out_specs=pl.BlockSpec((CB, 2 * D), lambda g: (g, 0)),
        out_shape=jax.ShapeDtypeStruct((VP, 2 * D), jnp.float32),
    )(weT)


def _tc_outformat(x):
    BB = 128
    LP = L // 2

    def body(in_ref, out_ref):
        x3 = in_ref[...].reshape(BB, LP, 2 * D)
        for lp in range(LP):
            zt = x3[:, lp, :].T
            out_ref[2 * lp, :, :] = zt[0:D, :]
            out_ref[2 * lp + 1, :, :] = zt[D:2 * D, :]

    return pl.pallas_call(
        body,
        grid=(B // BB,),
        in_specs=[pl.BlockSpec((BB * LP, 2 * D), lambda g: (g, 0))],
        out_specs=pl.BlockSpec((L, D, BB), lambda g: (0, 0, g)),
        out_shape=jax.ShapeDtypeStruct((L, D, B), jnp.float32),
    )(x)


def _sc_embed(we, idx):
    mesh = plsc.VectorSubcoreMesh(core_axis_name="c", subcore_axis_name="s")

    @functools.partial(
        pl.kernel,
        mesh=mesh,
        out_type=jax.ShapeDtypeStruct((B * L // 2, 2 * D), jnp.float32),
        scratch_types=[
            pltpu.VMEM((B_PER_W, L * K), jnp.int32),
            pltpu.VMEM((L * K, D), jnp.float32),
            pltpu.VMEM((L * K, D), jnp.float32),
            pltpu.VMEM((L // 2, 2 * D), jnp.float32),
            pltpu.VMEM((L // 2, 2 * D), jnp.float32),
            pltpu.SemaphoreType.DMA,
            pltpu.SemaphoreType.DMA,
            pltpu.SemaphoreType.DMA,
            pltpu.SemaphoreType.DMA,
        ],
        compiler_params=pltpu.CompilerParams(use_tc_tiling_on_sc=False),
    )
    def k(we_hbm, idx_hbm, out_hbm, idx_all, rows0, rows1, outv0, outv1,
          gsem0, gsem1, wsem0, wsem1):
        rows = (rows0, rows1)
        outv = (outv0, outv1)
        gsem = (gsem0, gsem1)
        wsem = (wsem0, wsem1)

        wid = lax.axis_index("s") * NC + lax.axis_index("c")
        base = wid * B_PER_W

        pltpu.sync_copy(idx_hbm.at[pl.ds(base, B_PER_W)], idx_all)

        def gather_copy(cc, b):
            return pltpu.make_async_copy(
                we_hbm.at[idx_all.at[cc]], rows[b], gsem[b])

        def out_copy(cc, b):
            return pltpu.make_async_copy(
                outv[b], out_hbm.at[pl.ds((base + cc) * (L // 2), L // 2)],
                wsem[b])

        gather_copy(0, 0).start()

        def step(cc, b):
            @pl.when(cc + 1 < B_PER_W)
            def _():
                gather_copy(cc + 1, 1 - b).start()

            gather_copy(cc, b).wait()

            @pl.when(cc >= 2)
            def _():
                out_copy(cc - 2, b).wait()

            rv = rows[b]
            ov = outv[b]

            def row_body(i2, _):
                for par in range(2):
                    r0 = K * (2 * i2 + par)
                    for v in range(D // 16):
                        so = pl.ds(par * D + v * 16, 16)
                        sr = pl.ds(v * 16, 16)
                        ov[i2, so] = rv[r0, sr] + rv[r0 + 1, sr] + rv[r0 + 2, sr]
                return 0

            lax.fori_loop(0, L // 2, row_body, 0)
            out_copy(cc, b).start()

        def pair_body(g, _):
            step(2 * g, 0)
            step(2 * g + 1, 1)
            return 0

        lax.fori_loop(0, B_PER_W // 2, pair_body, 0)
        out_copy(B_PER_W - 2, 0).wait()
        out_copy(B_PER_W - 1, 1).wait()

    return k(we, idx)


def kernel(inputs, we):
    we2 = _tc_detile(we.T).reshape(2 * VP, D)
    idx = (inputs.astype(jnp.int32) * 2).reshape(B, L * K)
    x = _sc_embed(we2, idx)
    ot = _tc_outformat(x)
    return jnp.transpose(ot, (2, 0, 1))


# speedup vs baseline: 1.3496x; 1.3496x over previous
import functools

import jax
import jax.numpy as jnp
from jax import lax
from jax.experimental import pallas as pl
from jax.experimental.pallas import tpu as pltpu
from jax.experimental.pallas import tpu_sc as plsc

B, L, K = 1024, 200, 3
D = 64
V = 1000518
NC, NS = 2, 16
NW = NC * NS
B_PER_W = B // NW


CB = 1024
G = -(-V // CB)
VP = G * CB


def _tc_detile(weT):
    def body(in_ref, out_ref):
        row = lax.broadcasted_iota(jnp.int32, (D, D), 0)
        col = lax.broadcasted_iota(jnp.int32, (D, D), 1)
        eye = jnp.where(row == col, 1.0, 0.0).astype(jnp.float32)
        out_ref[:, 0:D] = lax.dot_general(
            in_ref[...], eye, (((0,), (0,)), ((), ())),
            precision=lax.Precision.HIGHEST,
            preferred_element_type=jnp.float32)

    return pl.pallas_call(
        body,
        grid=(G,),
        in_specs=[pl.BlockSpec((D, CB), lambda g: (0, g))],
        out_specs=pl.BlockSpec((CB, 2 * D), lambda g: (g, 0)),
        out_shape=jax.ShapeDtypeStruct((VP, 2 * D), jnp.float32),
    )(weT)


def _tc_outformat(x):
    BB = 128
    LP = L // 2

    def body(in_ref, out_ref):
        x3 = in_ref[...].reshape(BB, LP, 2 * D)
        for lp in range(LP):
            zt = x3[:, lp, :].T
            out_ref[2 * lp, :, :] = zt[0:D, :]
            out_ref[2 * lp + 1, :, :] = zt[D:2 * D, :]

    return pl.pallas_call(
        body,
        grid=(B // BB,),
        in_specs=[pl.BlockSpec((BB * LP, 2 * D), lambda g: (g, 0))],
        out_specs=pl.BlockSpec((L, D, BB), lambda g: (0, 0, g)),
        out_shape=jax.ShapeDtypeStruct((L, D, B), jnp.float32),
    )(x)


def _sc_embed(we, idx):
    mesh = plsc.VectorSubcoreMesh(core_axis_name="c", subcore_axis_name="s")

    @functools.partial(
        pl.kernel,
        mesh=mesh,
        out_type=jax.ShapeDtypeStruct((B * L // 2, 2 * D), jnp.float32),
        scratch_types=[
            pltpu.VMEM((B_PER_W, L * K), jnp.int32),
            pltpu.VMEM((L * K, D), jnp.float32),
            pltpu.VMEM((L * K, D), jnp.float32),
            pltpu.VMEM((L // 2, 2 * D), jnp.float32),
            pltpu.VMEM((L // 2, 2 * D), jnp.float32),
            pltpu.SemaphoreType.DMA,
            pltpu.SemaphoreType.DMA,
            pltpu.SemaphoreType.DMA,
            pltpu.SemaphoreType.DMA,
        ],
        compiler_params=pltpu.CompilerParams(use_tc_tiling_on_sc=False),
    )
    def k(we_hbm, idx_hbm, out_hbm, idx_all, rows0, rows1, outv0, outv1,
          gsem0, gsem1, wsem0, wsem1):
        rows = (rows0, rows1)
        outv = (outv0, outv1)
        gsem = (gsem0, gsem1)
        wsem = (wsem0, wsem1)

        wid = lax.axis_index("s") * NC + lax.axis_index("c")
        base = wid * B_PER_W

        pltpu.sync_copy(idx_hbm.at[pl.ds(base, B_PER_W)], idx_all)

        def gather_copy(cc, b):
            return pltpu.make_async_copy(
                we_hbm.at[idx_all.at[cc]], rows[b], gsem[b])

        def out_copy(cc, b):
            return pltpu.make_async_copy(
                outv[b], out_hbm.at[pl.ds((base + cc) * (L // 2), L // 2)],
                wsem[b])

        gather_copy(0, 0).start()

        def step(cc, b):
            @pl.when(cc + 1 < B_PER_W)
            def _():
                gather_copy(cc + 1, 1 - b).start()

            gather_copy(cc, b).wait()

            @pl.when(cc >= 2)
            def _():
                out_copy(cc - 2, b).wait()

            rv = rows[b]
            ov = outv[b]

            def row_body(i2, _):
                for par in range(2):
                    r0 = K * (2 * i2 + par)
                    for v in range(D // 16):
                        so = pl.ds(par * D + v * 16, 16)
                        sr = pl.ds(v * 16, 16)
                        ov[i2, so] = rv[r0, sr] + rv[r0 + 1, sr] + rv[r0 + 2, sr]
                return 0

            lax.fori_loop(0, L // 2, row_body, 0)
            out_copy(cc, b).start()

        def pair_body(g, _):
            step(2 * g, 0)
            step(2 * g + 1, 1)
            return 0

        lax.fori_loop(0, B_PER_W // 2, pair_body, 0)
        out_copy(B_PER_W - 2, 0).wait()
        out_copy(B_PER_W - 1, 1).wait()

    return k(we, idx)


def kernel(inputs, we):
    idx = inputs.astype(jnp.int32).reshape(B, L * K)
    x = _sc_embed(we, idx)
    ot = _tc_outformat(x)
    return jnp.transpose(ot, (2, 0, 1))
